# Initial kernel scaffold; baseline (speedup 1.0000x reference)
import jax
import jax.numpy as jnp
from jax.experimental import pallas as pl


def kernel(x, edge_index, W1, b1, W2, b2):
    raise NotImplementedError("write your pallas kernel here")


# speedup vs baseline: 3.5654x; 3.5654x over previous
import functools

import jax
import jax.numpy as jnp
import numpy as np
from jax import lax
from jax.experimental import pallas as pl
from jax.experimental.pallas import tpu as pltpu
from jax.experimental.pallas import tpu_sc as plsc

N = 10000
E = 320000
K = 8

NC = 2
NS = 16
NW = NC * NS
CHUNK = 128
CPW = 80
EPAD = NW * CPW * CHUNK
GROW = N
NA = N + 112
RPT = NA // NS
LW = 128

F1 = 128
FH = 64
FO = 64
QW = K * FH

_CHEB = np.zeros((K, K), np.float64)
_CHEB[0, 0] = 1.0
_CHEB[1, 1] = 1.0
for _k in range(2, K):
    _CHEB[_k, 1:] += 2.0 * _CHEB[_k - 1, :-1]
    _CHEB[_k, :] -= _CHEB[_k - 2, :]


_MESH = plsc.VectorSubcoreMesh(core_axis_name="c", subcore_axis_name="s",
                               num_cores=NC, num_subcores=NS)


def _sc_deg_body(srcdeg_hbm, ones_hbm, zeros_hbm, out_hbm, acc_sh, ones_v,
                 idx_v):
    c = lax.axis_index("c")
    s = lax.axis_index("s")
    wid = c * NS + s
    row0 = s * RPT
    pltpu.sync_copy(zeros_hbm.at[pl.ds(row0, RPT)], acc_sh.at[pl.ds(row0, RPT)])
    pltpu.sync_copy(ones_hbm, ones_v)
    pltpu.sync_copy(srcdeg_hbm.at[wid], idx_v)
    plsc.subcore_barrier()

    def body(j, carry):
        pltpu.sync_copy(ones_v, acc_sh.at[idx_v.at[j]], add=True)
        return carry

    lax.fori_loop(0, CPW, body, 0)
    plsc.subcore_barrier()
    pltpu.sync_copy(acc_sh.at[pl.ds(row0, RPT)], out_hbm.at[c, pl.ds(row0, RPT)])


_sc_deg = pl.kernel(
    _sc_deg_body,
    out_type=jax.ShapeDtypeStruct((NC, NA, LW), jnp.float32),
    mesh=_MESH,
    scratch_types=[
        pltpu.VMEM_SHARED((NA, LW), jnp.float32),
        pltpu.VMEM((CHUNK, LW), jnp.float32),
        pltpu.VMEM((CPW, CHUNK), jnp.int32),
    ],
)


def _sc_prop_body(t_hbm, src_hbm, dst_hbm, zeros_hbm, out_hbm, acc_sh, src_v,
                  dst_v, rows_v, sem):
    c = lax.axis_index("c")
    s = lax.axis_index("s")
    wid = c * NS + s
    row0 = s * RPT
    pltpu.sync_copy(zeros_hbm.at[pl.ds(row0, RPT)], acc_sh.at[pl.ds(row0, RPT)])
    pltpu.sync_copy(src_hbm.at[wid], src_v)
    pltpu.sync_copy(dst_hbm.at[wid], dst_v)
    plsc.subcore_barrier()

    def body(j, carry):
        pltpu.async_copy(t_hbm.at[src_v.at[j]], rows_v, sem).wait()
        pltpu.sync_copy(rows_v, acc_sh.at[dst_v.at[j]], add=True)
        return carry

    lax.fori_loop(0, CPW, body, 0)
    plsc.subcore_barrier()
    pltpu.sync_copy(acc_sh.at[pl.ds(row0, RPT)], out_hbm.at[c, pl.ds(row0, RPT)])


_sc_prop = pl.kernel(
    _sc_prop_body,
    out_type=jax.ShapeDtypeStruct((NC, NA, LW), jnp.float32),
    mesh=_MESH,
    scratch_types=[
        pltpu.VMEM_SHARED((NA, LW), jnp.float32),
        pltpu.VMEM((CPW, CHUNK), jnp.int32),
        pltpu.VMEM((CPW, CHUNK), jnp.int32),
        pltpu.VMEM((CHUNK, LW), jnp.float32),
        pltpu.SemaphoreType.DMA,
    ],
)


def _tc_prep_body(degp_ref, dinv_ref, d2_ref):
    deg = degp_ref[0, :, 0:1] + degp_ref[1, :, 0:1]
    dinv = jnp.where(deg > 0.0, lax.rsqrt(jnp.maximum(deg, 1e-30)), 0.0)
    dinv_ref[...] = dinv
    d2_ref[...] = dinv * dinv


_RB = 1000


def _tc_prep(degp):
    return pl.pallas_call(
        _tc_prep_body,
        grid=(NA // RPT,),
        in_specs=[pl.BlockSpec((NC, RPT, LW), lambda i: (0, i, 0))],
        out_specs=(pl.BlockSpec((RPT, 1), lambda i: (i, 0)),
                   pl.BlockSpec((RPT, 1), lambda i: (i, 0))),
        out_shape=(jax.ShapeDtypeStruct((NA, 1), jnp.float32),
                   jax.ShapeDtypeStruct((NA, 1), jnp.float32)),
    )(degp)


def _tc_mm_body(x_ref, c_ref, b_ref, dinv_ref, q_ref, *, do_relu):
    xb = x_ref[...]
    if do_relu:
        xb = jnp.maximum(xb, 0.0)
    mm = jnp.dot(xb, c_ref[...], preferred_element_type=jnp.float32,
                 precision=lax.Precision.HIGHEST)
    col = lax.broadcasted_iota(jnp.int32, (_RB, QW), 1)
    scale = jnp.where(col < FH, 1.0, dinv_ref[...])
    q_ref[...] = mm * scale + b_ref[...]


def _tc_mm(x, ccat, bpad, dinv, do_relu):
    fin = x.shape[1]
    return pl.pallas_call(
        functools.partial(_tc_mm_body, do_relu=do_relu),
        grid=(N // _RB,),
        in_specs=[
            pl.BlockSpec((_RB, fin), lambda i: (i, 0)),
            pl.BlockSpec((fin, QW), lambda i: (0, 0)),
            pl.BlockSpec((1, QW), lambda i: (0, 0)),
            pl.BlockSpec((_RB, 1), lambda i: (i, 0)),
        ],
        out_specs=pl.BlockSpec((_RB, QW), lambda i: (i, 0)),
        out_shape=jax.ShapeDtypeStruct((N, QW), jnp.float32),
    )(x, ccat, bpad, dinv)


def _tc_combine_mid_body(q_ref, a_ref, dvec_ref, o_ref):
    acc = a_ref[0] + a_ref[1]
    s = q_ref[...] - dvec_ref[...] * acc[:, :FH]
    o_ref[...] = jnp.concatenate([s, s], axis=1)


def _tc_combine_fin_body(q_ref, a_ref, dvec_ref, o_ref):
    acc = a_ref[0] + a_ref[1]
    o_ref[...] = q_ref[...] - dvec_ref[...] * acc[:, :FH]


def _tc_combine(q, j, accp, dvec, final):
    qj = lax.slice(q, (0, j * FH), (N, (j + 1) * FH))
    body = _tc_combine_fin_body if final else _tc_combine_mid_body
    ow = FH if final else LW
    return pl.pallas_call(
        body,
        grid=(N // _RB,),
        in_specs=[
            pl.BlockSpec((_RB, FH), lambda i: (i, 0)),
            pl.BlockSpec((NC, _RB, LW), lambda i: (0, i, 0)),
            pl.BlockSpec((_RB, 1), lambda i: (i, 0)),
        ],
        out_specs=pl.BlockSpec((_RB, ow), lambda i: (i, 0)),
        out_shape=jax.ShapeDtypeStruct((N, ow), jnp.float32),
    )(qj, accp, dvec)


def _pad_edges(v, fill):
    return jnp.concatenate(
        [v, jnp.full((EPAD - E,), fill, jnp.int32)]).reshape(NW, CPW, CHUNK)


def _layer(s_table, q, srcg, dstg, zeros, dinv, d2):
    for j in range(6, 0, -1):
        accp = _sc_prop(s_table, srcg, dstg, zeros)
        s_table = _tc_combine(q, j, accp, d2, final=False)
    accp = _sc_prop(s_table, srcg, dstg, zeros)
    return _tc_combine(q, 0, accp, dinv, final=True)


def kernel(x, edge_index, W1, b1, W2, b2):
    src = edge_index[0]
    dst = edge_index[1]
    self_e = src == dst

    srcg = _pad_edges(src, 0)
    dstg = _pad_edges(jnp.where(self_e, GROW, dst), GROW)
    srcdeg = _pad_edges(jnp.where(self_e, GROW, src), GROW)

    cheb = jnp.asarray(_CHEB, jnp.float32)
    c1 = jnp.einsum("kj,kio->jio", cheb, W1)
    ccat1 = jnp.transpose(c1, (1, 0, 2)).reshape(F1, QW)
    c2 = jnp.einsum("kj,kio->jio", cheb, W2)
    ccat2 = jnp.transpose(c2, (1, 0, 2)).reshape(FH, QW)
    bpad1 = jnp.concatenate([b1, jnp.zeros((QW - FH,), jnp.float32)])[None, :]
    bpad2 = jnp.concatenate([b2, jnp.zeros((QW - FO,), jnp.float32)])[None, :]

    zeros = jnp.zeros((NA, LW), jnp.float32)
    ones = jnp.ones((CHUNK, LW), jnp.float32)

    degp = _sc_deg(srcdeg, ones, zeros)
    dinv, d2 = _tc_prep(degp)

    q1 = _tc_mm(x, ccat1, bpad1, dinv, do_relu=False)
    s7 = lax.slice(q1, (0, 7 * FH), (N, QW))
    s7 = jnp.concatenate([s7, s7], axis=1)
    h = _layer(s7, q1, srcg, dstg, zeros, dinv, d2)

    q2 = _tc_mm(h, ccat2, bpad2, dinv, do_relu=True)
    s7b = lax.slice(q2, (0, 7 * FH), (N, QW))
    s7b = jnp.concatenate([s7b, s7b], axis=1)
    return _layer(s7b, q2, srcg, dstg, zeros, dinv, d2)


# speedup vs baseline: 4.0306x; 1.1305x over previous
import functools

import jax
import jax.numpy as jnp
import numpy as np
from jax import lax
from jax.experimental import pallas as pl
from jax.experimental.pallas import tpu as pltpu
from jax.experimental.pallas import tpu_sc as plsc

N = 10000
E = 320000
K = 8

NC = 2
NS = 16
NW = NC * NS
CHUNK = 128
CPW = 80
EPAD = NW * CPW * CHUNK
GROW = N
NA = N + 112
RPT = NA // NS
LW = 128

F1 = 128
FH = 64
FO = 64
QW = K * FH

_CHEB = np.zeros((K, K), np.float64)
_CHEB[0, 0] = 1.0
_CHEB[1, 1] = 1.0
for _k in range(2, K):
    _CHEB[_k, 1:] += 2.0 * _CHEB[_k - 1, :-1]
    _CHEB[_k, :] -= _CHEB[_k - 2, :]


_MESH = plsc.VectorSubcoreMesh(core_axis_name="c", subcore_axis_name="s",
                               num_cores=NC, num_subcores=NS)


def _sc_deg_body(srcdeg_hbm, ones_hbm, zeros_hbm, out_hbm, acc_sh, ones_v,
                 idx_v, sem):
    c = lax.axis_index("c")
    s = lax.axis_index("s")
    wid = c * NS + s
    row0 = s * RPT
    pltpu.sync_copy(zeros_hbm.at[pl.ds(row0, RPT)], acc_sh.at[pl.ds(row0, RPT)])
    pltpu.sync_copy(ones_hbm, ones_v)
    pltpu.sync_copy(srcdeg_hbm.at[wid], idx_v)
    plsc.subcore_barrier()

    def body(j, carry):
        pltpu.async_copy(ones_v, acc_sh.at[idx_v.at[j]], sem, add=True)
        return carry

    lax.fori_loop(0, CPW, body, 0)

    def drain(j, carry):
        pltpu.make_async_copy(ones_v, acc_sh.at[idx_v.at[j]], sem).wait()
        return carry

    lax.fori_loop(0, CPW, drain, 0)
    plsc.subcore_barrier()
    pltpu.sync_copy(acc_sh.at[pl.ds(row0, RPT)], out_hbm.at[c, pl.ds(row0, RPT)])


_sc_deg = pl.kernel(
    _sc_deg_body,
    out_type=jax.ShapeDtypeStruct((NC, NA, LW), jnp.float32),
    mesh=_MESH,
    scratch_types=[
        pltpu.VMEM_SHARED((NA, LW), jnp.float32),
        pltpu.VMEM((CHUNK, LW), jnp.float32),
        pltpu.VMEM((CPW, CHUNK), jnp.int32),
        pltpu.SemaphoreType.DMA,
    ],
)


_SG = 8
_NSG = CPW // _SG


def _sc_prop_body(t_hbm, src_hbm, dst_hbm, zeros_hbm, out_hbm, acc_sh, src_v,
                  dring, rows_v, gsemA, gsemB, ssemA, ssemB, isemA, isemB):
    c = lax.axis_index("c")
    s = lax.axis_index("s")
    wid = c * NS + s
    row0 = s * RPT
    pltpu.sync_copy(zeros_hbm.at[pl.ds(row0, RPT)], acc_sh.at[pl.ds(row0, RPT)])
    pltpu.sync_copy(src_hbm.at[wid], src_v)
    plsc.subcore_barrier()

    def gather(j, p, sem):
        pltpu.async_copy(t_hbm.at[src_v.at[j]], rows_v.at[p], sem)

    def wait_gather(j, p, sem):
        pltpu.make_async_copy(t_hbm.at[src_v.at[j]], rows_v.at[p], sem).wait()

    def scat(rp, k, p, sem):
        pltpu.async_copy(rows_v.at[p], acc_sh.at[dring.at[rp, k]], sem,
                         add=True)

    def wait_scat(rp, k, p, sem):
        pltpu.make_async_copy(rows_v.at[p], acc_sh.at[dring.at[rp, k]],
                              sem).wait()

    def load_ring(sg, rp, isem):
        pltpu.async_copy(dst_hbm.at[wid, pl.ds(sg * _SG, _SG)], dring.at[rp],
                         isem)

    def wait_ring(sg, rp, isem):
        pltpu.make_async_copy(dst_hbm.at[wid, pl.ds(sg * _SG, _SG)],
                              dring.at[rp], isem).wait()

    def supergroup(sg, rp, isem, gsems, ssems, ring_prefetch, kmax=_SG):
        wait_ring(sg, rp, isem)
        for k in range(_SG):
            j = sg * _SG + k
            p = k % 2
            wait_gather(j, p, gsems[p])
            scat(rp, k, p, ssems[p])
            wait_scat(rp, k, p, ssems[p])
            if k < kmax:
                gather(j + 2, p, gsems[p])
        if ring_prefetch:
            load_ring(sg + 2, rp, isem)

    load_ring(0, 0, isemA)
    load_ring(1, 1, isemB)
    gather(0, 0, gsemA)
    gather(1, 1, gsemB)

    gsems = (gsemA, gsemB)
    ssems = (ssemA, ssemB)

    def body(i, carry):
        supergroup(2 * i, 0, isemA, gsems, ssems, True)
        supergroup(2 * i + 1, 1, isemB, gsems, ssems, True)
        return carry

    lax.fori_loop(0, (_NSG - 2) // 2, body, 0)
    supergroup(_NSG - 2, 0, isemA, gsems, ssems, False)
    supergroup(_NSG - 1, 1, isemB, gsems, ssems, False, kmax=_SG - 2)

    plsc.subcore_barrier()
    pltpu.sync_copy(acc_sh.at[pl.ds(row0, RPT)], out_hbm.at[c, pl.ds(row0, RPT)])


_sc_prop = pl.kernel(
    _sc_prop_body,
    out_type=jax.ShapeDtypeStruct((NC, NA, LW), jnp.float32),
    mesh=_MESH,
    scratch_types=[
        pltpu.VMEM_SHARED((NA, LW), jnp.float32),
        pltpu.VMEM((CPW, CHUNK), jnp.int32),
        pltpu.VMEM((2, _SG, CHUNK), jnp.int32),
        pltpu.VMEM((2, CHUNK, LW), jnp.float32),
        pltpu.SemaphoreType.DMA,
        pltpu.SemaphoreType.DMA,
        pltpu.SemaphoreType.DMA,
        pltpu.SemaphoreType.DMA,
        pltpu.SemaphoreType.DMA,
        pltpu.SemaphoreType.DMA,
    ],
)


def _tc_prep_body(degp_ref, dinv_ref, d2_ref):
    deg = degp_ref[0, :, 0:1] + degp_ref[1, :, 0:1]
    dinv = jnp.where(deg > 0.0, lax.rsqrt(jnp.maximum(deg, 1e-30)), 0.0)
    dinv_ref[...] = dinv
    d2_ref[...] = dinv * dinv


_RB = 1000


def _tc_prep(degp):
    return pl.pallas_call(
        _tc_prep_body,
        grid=(NA // RPT,),
        in_specs=[pl.BlockSpec((NC, RPT, LW), lambda i: (0, i, 0))],
        out_specs=(pl.BlockSpec((RPT, 1), lambda i: (i, 0)),
                   pl.BlockSpec((RPT, 1), lambda i: (i, 0))),
        out_shape=(jax.ShapeDtypeStruct((NA, 1), jnp.float32),
                   jax.ShapeDtypeStruct((NA, 1), jnp.float32)),
    )(degp)


def _tc_mm_body(x_ref, c_ref, b_ref, dinv_ref, q_ref, *, do_relu):
    xb = x_ref[...]
    if do_relu:
        xb = jnp.maximum(xb, 0.0)
    mm = jnp.dot(xb, c_ref[...], preferred_element_type=jnp.float32,
                 precision=lax.Precision.HIGHEST)
    col = lax.broadcasted_iota(jnp.int32, (_RB, QW), 1)
    scale = jnp.where(col < FH, 1.0, dinv_ref[...])
    q_ref[...] = mm * scale + b_ref[...]


def _tc_mm(x, ccat, bpad, dinv, do_relu):
    fin = x.shape[1]
    return pl.pallas_call(
        functools.partial(_tc_mm_body, do_relu=do_relu),
        grid=(N // _RB,),
        in_specs=[
            pl.BlockSpec((_RB, fin), lambda i: (i, 0)),
            pl.BlockSpec((fin, QW), lambda i: (0, 0)),
            pl.BlockSpec((1, QW), lambda i: (0, 0)),
            pl.BlockSpec((_RB, 1), lambda i: (i, 0)),
        ],
        out_specs=pl.BlockSpec((_RB, QW), lambda i: (i, 0)),
        out_shape=jax.ShapeDtypeStruct((N, QW), jnp.float32),
    )(x, ccat, bpad, dinv)


def _tc_combine_mid_body(q_ref, a_ref, dvec_ref, o_ref):
    acc = a_ref[0] + a_ref[1]
    s = q_ref[...] - dvec_ref[...] * acc[:, :FH]
    o_ref[...] = jnp.concatenate([s, s], axis=1)


def _tc_combine_fin_body(q_ref, a_ref, dvec_ref, o_ref):
    acc = a_ref[0] + a_ref[1]
    o_ref[...] = q_ref[...] - dvec_ref[...] * acc[:, :FH]


def _tc_combine(q, j, accp, dvec, final):
    qj = lax.slice(q, (0, j * FH), (N, (j + 1) * FH))
    body = _tc_combine_fin_body if final else _tc_combine_mid_body
    ow = FH if final else LW
    return pl.pallas_call(
        body,
        grid=(N // _RB,),
        in_specs=[
            pl.BlockSpec((_RB, FH), lambda i: (i, 0)),
            pl.BlockSpec((NC, _RB, LW), lambda i: (0, i, 0)),
            pl.BlockSpec((_RB, 1), lambda i: (i, 0)),
        ],
        out_specs=pl.BlockSpec((_RB, ow), lambda i: (i, 0)),
        out_shape=jax.ShapeDtypeStruct((N, ow), jnp.float32),
    )(qj, accp, dvec)


def _pad_edges(v, fill):
    return jnp.concatenate(
        [v, jnp.full((EPAD - E,), fill, jnp.int32)]).reshape(NW, CPW, CHUNK)


def _layer(s_table, q, srcg, dstg, zeros, dinv, d2):
    for j in range(6, 0, -1):
        accp = _sc_prop(s_table, srcg, dstg, zeros)
        s_table = _tc_combine(q, j, accp, d2, final=False)
    accp = _sc_prop(s_table, srcg, dstg, zeros)
    return _tc_combine(q, 0, accp, dinv, final=True)


def kernel(x, edge_index, W1, b1, W2, b2):
    src = edge_index[0]
    dst = edge_index[1]
    self_e = src == dst

    srcg = _pad_edges(src, 0)
    dstg = _pad_edges(jnp.where(self_e, GROW, dst), GROW)
    srcdeg = _pad_edges(jnp.where(self_e, GROW, src), GROW)

    cheb = jnp.asarray(_CHEB, jnp.float32)
    c1 = jnp.einsum("kj,kio->jio", cheb, W1)
    ccat1 = jnp.transpose(c1, (1, 0, 2)).reshape(F1, QW)
    c2 = jnp.einsum("kj,kio->jio", cheb, W2)
    ccat2 = jnp.transpose(c2, (1, 0, 2)).reshape(FH, QW)
    bpad1 = jnp.concatenate([b1, jnp.zeros((QW - FH,), jnp.float32)])[None, :]
    bpad2 = jnp.concatenate([b2, jnp.zeros((QW - FO,), jnp.float32)])[None, :]

    zeros = jnp.zeros((NA, LW), jnp.float32)
    ones = jnp.ones((CHUNK, LW), jnp.float32)

    degp = _sc_deg(srcdeg, ones, zeros)
    dinv, d2 = _tc_prep(degp)

    q1 = _tc_mm(x, ccat1, bpad1, dinv, do_relu=False)
    s7 = lax.slice(q1, (0, 7 * FH), (N, QW))
    s7 = jnp.concatenate([s7, s7], axis=1)
    h = _layer(s7, q1, srcg, dstg, zeros, dinv, d2)

    q2 = _tc_mm(h, ccat2, bpad2, dinv, do_relu=True)
    s7b = lax.slice(q2, (0, 7 * FH), (N, QW))
    s7b = jnp.concatenate([s7b, s7b], axis=1)
    return _layer(s7b, q2, srcg, dstg, zeros, dinv, d2)
